# baseline (speedup 1.0000x reference)
import jax
import jax.numpy as jnp
from jax import lax
from jax.experimental import pallas as pl
from jax.experimental.pallas import tpu as pltpu

_BF = 128


def _bcast_kernel(dummy_ref, pos_ref, out_hbm, stage, sem):
    del dummy_ref
    c, b, s = out_hbm.shape
    for j in range(c):
        row = pos_ref[pl.ds(j, 1), :]
        stage[j] = jnp.broadcast_to(row, (_BF, s))
    n = b // _BF
    for j in range(c):
        for i in range(n):
            pltpu.make_async_copy(
                stage.at[pl.ds(j, 1)],
                out_hbm.at[pl.ds(j, 1), pl.ds(i * _BF, _BF), :],
                sem,
            ).start()
    for j in range(c):
        for i in range(n):
            pltpu.make_async_copy(
                stage.at[pl.ds(j, 1)],
                out_hbm.at[pl.ds(j, 1), pl.ds(i * _BF, _BF), :],
                sem,
            ).wait()


def kernel(dummy, position_ids):
    b = dummy.shape[0]
    _, c, one, s = position_ids.shape
    table = lax.bitcast_convert_type(position_ids.reshape(c, s), jnp.bfloat16)
    idx2d = dummy.reshape(1, b)
    out = pl.pallas_call(
        _bcast_kernel,
        in_specs=[
            pl.BlockSpec((1, b), lambda: (0, 0)),
            pl.BlockSpec((c, s), lambda: (0, 0)),
        ],
        out_specs=pl.BlockSpec(memory_space=pl.ANY),
        out_shape=jax.ShapeDtypeStruct((c, b, s), jnp.bfloat16),
        scratch_shapes=[
            pltpu.VMEM((c, _BF, s), jnp.bfloat16),
            pltpu.SemaphoreType.DMA,
        ],
    )(idx2d, table)
    out16 = lax.bitcast_convert_type(out, position_ids.dtype)
    return jnp.transpose(out16, (1, 0, 2)).reshape(b, c, one, s)
